# baseline (speedup 1.0000x reference)
import functools

import jax
import jax.numpy as jnp
from jax import lax
from jax.experimental import pallas as pl
from jax.experimental.pallas import tpu as pltpu
from jax.experimental.pallas import tpu_sc as plsc

VOCAB = 100000
D = 128
NUM_CORES = 2
NUM_SUBCORES = 16
NW = NUM_CORES * NUM_SUBCORES
CHUNK = 128
SUPER = 256
SUBS = ((0, 128), (128, 128))


@functools.partial(jax.jit, static_argnums=(2,))
def _lookup(flat_idx, embeddings, n):
    b_per_w = n // NW
    n_super = b_per_w // SUPER
    mesh = plsc.VectorSubcoreMesh(core_axis_name="c", subcore_axis_name="s")

    assert n_super % 2 == 1 and b_per_w % SUPER == 0

    @functools.partial(
        pl.kernel,
        mesh=mesh,
        out_type=jax.ShapeDtypeStruct((n, D), jnp.float32),
        scratch_types=[
            pltpu.VMEM((b_per_w,), jnp.int32),
            pltpu.VMEM((SUPER, D), jnp.float32),
            pltpu.VMEM((SUPER, D), jnp.float32),
            pltpu.SemaphoreType.DMA,
            pltpu.SemaphoreType.DMA,
        ],
    )
    def k(idx_hbm, table_hbm, out_hbm, idx_v, rows0, rows1, sem0, sem1):
        wid = lax.axis_index("s") * NUM_CORES + lax.axis_index("c")
        base = wid * b_per_w
        pltpu.sync_copy(idx_hbm.at[pl.ds(base, b_per_w)], idx_v)

        bufs = (rows0, rows1)
        sems = (sem0, sem1)

        def streams(s, buf, sem):
            return [
                pltpu.make_async_copy(
                    table_hbm.at[idx_v.at[pl.ds(s * SUPER + off, sz)]],
                    buf.at[pl.ds(off, sz)],
                    sem,
                )
                for off, sz in SUBS
            ]

        def fire(s, buf, sem):
            for st in streams(s, buf, sem):
                st.start()

        fire(0, rows0, sem0)

        def body(g, carry):
            for j in range(2):
                s = 2 * g + j
                buf, sem = bufs[j], sems[j]
                nbuf, nsem = bufs[1 - j], sems[1 - j]
                for st in streams(s, buf, sem):
                    st.wait()
                fire(s + 1, nbuf, nsem)
                pltpu.sync_copy(buf, out_hbm.at[pl.ds(base + s * SUPER, SUPER)])
            return carry

        lax.fori_loop(0, n_super // 2, body, 0)
        last = n_super - 1
        for st in streams(last, rows0, sem0):
            st.wait()
        pltpu.sync_copy(rows0, out_hbm.at[pl.ds(base + last * SUPER, SUPER)])

    return k(flat_idx, embeddings)


def kernel(inputs, embeddings):
    shape = inputs.shape
    flat = jnp.reshape(inputs, (-1,)).astype(jnp.int32)
    out = _lookup(flat, embeddings, flat.shape[0])
    return jnp.reshape(out, tuple(shape) + (D,))


# speedup vs baseline: 1.7488x; 1.7488x over previous
import functools

import jax
import jax.numpy as jnp
from jax import lax
from jax.experimental import pallas as pl
from jax.experimental.pallas import tpu as pltpu
from jax.experimental.pallas import tpu_sc as plsc

D = 128
NUM_CORES = 2
NUM_SUBCORES = 16
NW = NUM_CORES * NUM_SUBCORES
G = 4


@jax.jit
def _lookup(idx, embeddings):
    nb, row = idx.shape
    per_tile = nb // NW
    n_groups = per_tile // G
    assert per_tile % G == 0 and n_groups % 2 == 0 and n_groups >= 4
    mesh = plsc.VectorSubcoreMesh(core_axis_name="c", subcore_axis_name="s")

    @functools.partial(
        pl.kernel,
        mesh=mesh,
        out_type=jax.ShapeDtypeStruct((nb, row, D), jnp.float32),
        scratch_types=[
            pltpu.VMEM((per_tile, row), jnp.int32),
            pltpu.VMEM((G, row, D), jnp.float32),
            pltpu.VMEM((G, row, D), jnp.float32),
            pltpu.SemaphoreType.DMA,
            pltpu.SemaphoreType.DMA,
        ],
    )
    def k(idx_hbm, table_hbm, out_hbm, idx_v, buf0, buf1, sem0, sem1):
        wid = lax.axis_index("s") * NUM_CORES + lax.axis_index("c")
        b0 = wid * per_tile
        pltpu.sync_copy(idx_hbm.at[pl.ds(b0, per_tile)], idx_v)

        bufs = (buf0, buf1)
        sems = (sem0, sem1)

        def streams(g, buf, sem):
            return [
                pltpu.make_async_copy(
                    table_hbm.at[idx_v.at[g * G + i]], buf.at[i], sem
                )
                for i in range(G)
            ]

        def fire(g, buf, sem):
            for st in streams(g, buf, sem):
                st.start()

        def drain(g, buf, sem):
            for st in streams(g, buf, sem):
                st.wait()

        def flush(g, buf):
            pltpu.sync_copy(buf, out_hbm.at[pl.ds(b0 + g * G, G)])

        fire(0, buf0, sem0)

        def body(h, carry):
            for j in range(2):
                g = 2 * h + j
                buf, sem = bufs[j], sems[j]
                drain(g, buf, sem)
                fire(g + 1, bufs[1 - j], sems[1 - j])
                flush(g, buf)
            return carry

        lax.fori_loop(0, n_groups // 2 - 1, body, 0)
        g = n_groups - 2
        drain(g, buf0, sem0)
        fire(g + 1, buf1, sem1)
        flush(g, buf0)
        drain(g + 1, buf1, sem1)
        flush(g + 1, buf1)

    return k(idx, embeddings)


def kernel(inputs, embeddings):
    return _lookup(inputs.astype(jnp.int32), embeddings)
